# baseline (speedup 1.0000x reference)
import functools

import jax
import jax.numpy as jnp
from jax import lax
from jax.experimental import pallas as pl
from jax.experimental.pallas import tpu as pltpu
from jax.experimental.pallas import tpu_sc as plsc

_EPS = 0.1
_B, _H, _D = 8, 512, 512
_N = _B * _H * _D
_NC, _NS = 2, 16
_NW = _NC * _NS
_PER_W = _N // _NW
_CHUNK = 16384
_NCHUNK = _PER_W // _CHUNK
_LANES = 16
_UNROLL = 16


_ROWS_PER_CHUNK = _CHUNK // _D
_ROWS_PER_TILE = _PER_W // _D
_TILES_PER_B = _H // _ROWS_PER_TILE


def _reduce_chunk(buf, accs):
    def body(i, carry):
        a0, a1, a2, a3 = carry
        for u in range(0, _D // _LANES, 4):
            a0 = a0 + buf[i, pl.ds((u + 0) * _LANES, _LANES)]
            a1 = a1 + buf[i, pl.ds((u + 1) * _LANES, _LANES)]
            a2 = a2 + buf[i, pl.ds((u + 2) * _LANES, _LANES)]
            a3 = a3 + buf[i, pl.ds((u + 3) * _LANES, _LANES)]
        return (a0, a1, a2, a3)

    return lax.fori_loop(0, _ROWS_PER_CHUNK, body, accs)


def _partial_sums(gt):
    mesh = plsc.VectorSubcoreMesh(core_axis_name="c", subcore_axis_name="s")

    @functools.partial(
        pl.kernel,
        out_type=jax.ShapeDtypeStruct((_NW, _LANES), jnp.int32),
        mesh=mesh,
        scratch_types=[
            pltpu.VMEM((4, _ROWS_PER_CHUNK, _D), jnp.int32),
            pltpu.VMEM((_LANES,), jnp.int32),
            pltpu.SemaphoreType.DMA,
            pltpu.SemaphoreType.DMA,
            pltpu.SemaphoreType.DMA,
            pltpu.SemaphoreType.DMA,
        ],
    )
    def k(x_hbm, out_hbm, buf, stage, sem0, sem1, sem2, sem3):
        c = lax.axis_index("c")
        s = lax.axis_index("s")
        wid = s * _NC + c
        b = wid // _TILES_PER_B
        r0 = (wid % _TILES_PER_B) * _ROWS_PER_TILE

        sems = (sem0, sem1, sem2, sem3)
        zero = jnp.zeros((_LANES,), jnp.int32)
        accs = (zero, zero, zero, zero)

        copies = [
            pltpu.async_copy(
                x_hbm.at[b, pl.ds(r0 + j * _ROWS_PER_CHUNK,
                                  _ROWS_PER_CHUNK)],
                buf.at[j], sems[j])
            for j in range(_NCHUNK)
        ]
        for j in range(_NCHUNK):
            copies[j].wait()
            accs = _reduce_chunk(buf.at[j], accs)

        stage[...] = (accs[0] + accs[1]) + (accs[2] + accs[3])
        pltpu.sync_copy(stage, out_hbm.at[wid])

    return k(gt)


def kernel(gt):
    partials = _partial_sums(gt)
    count1 = jnp.sum(partials)
    sizes = jnp.stack(
        [jnp.int32(_N) - count1, count1]).astype(jnp.float32)
    lowbound = (sizes * (1.0 - _EPS)).astype(jnp.int32)
    highbound = (sizes * (1.0 + _EPS)).astype(jnp.int32)
    return jnp.stack((lowbound, highbound))


# speedup vs baseline: 1.0695x; 1.0695x over previous
import functools

import jax
import jax.numpy as jnp
from jax import lax
from jax.experimental import pallas as pl
from jax.experimental.pallas import tpu as pltpu
from jax.experimental.pallas import tpu_sc as plsc

_EPS = 0.1
_B, _H, _D = 8, 512, 512
_N = _B * _H * _D

_SC_B0 = 4
_SC_N = (_B - _SC_B0) * _H * _D
_NC, _NS = 2, 16
_NW = _NC * _NS
_PER_W = _SC_N // _NW
_LANES = 16
_ROWS_PER_TILE = _PER_W // _D
_TILES_PER_B = _H // _ROWS_PER_TILE
_NCHUNK = 4
_ROWS_PER_CHUNK = _ROWS_PER_TILE // _NCHUNK


def _reduce_chunk(buf, accs):
    def body(i, carry):
        a0, a1, a2, a3 = carry
        for u in range(0, _D // _LANES, 4):
            a0 = a0 + buf[i, pl.ds((u + 0) * _LANES, _LANES)]
            a1 = a1 + buf[i, pl.ds((u + 1) * _LANES, _LANES)]
            a2 = a2 + buf[i, pl.ds((u + 2) * _LANES, _LANES)]
            a3 = a3 + buf[i, pl.ds((u + 3) * _LANES, _LANES)]
        return (a0, a1, a2, a3)

    return lax.fori_loop(0, _ROWS_PER_CHUNK, body, accs)


def _sc_partial_sums(gt):
    mesh = plsc.VectorSubcoreMesh(core_axis_name="c", subcore_axis_name="s")

    @functools.partial(
        pl.kernel,
        out_type=jax.ShapeDtypeStruct((_NW, _LANES), jnp.int32),
        mesh=mesh,
        scratch_types=[
            pltpu.VMEM((_NCHUNK, _ROWS_PER_CHUNK, _D), jnp.int32),
            pltpu.VMEM((_LANES,), jnp.int32),
            pltpu.SemaphoreType.DMA,
            pltpu.SemaphoreType.DMA,
            pltpu.SemaphoreType.DMA,
            pltpu.SemaphoreType.DMA,
        ],
    )
    def k(x_hbm, out_hbm, buf, stage, sem0, sem1, sem2, sem3):
        c = lax.axis_index("c")
        s = lax.axis_index("s")
        wid = s * _NC + c
        b = _SC_B0 + wid // _TILES_PER_B
        r0 = (wid % _TILES_PER_B) * _ROWS_PER_TILE

        sems = (sem0, sem1, sem2, sem3)
        zero = jnp.zeros((_LANES,), jnp.int32)
        accs = (zero, zero, zero, zero)

        copies = [
            pltpu.async_copy(
                x_hbm.at[b, pl.ds(r0 + j * _ROWS_PER_CHUNK,
                                  _ROWS_PER_CHUNK)],
                buf.at[j], sems[j])
            for j in range(_NCHUNK)
        ]
        for j in range(_NCHUNK):
            copies[j].wait()
            accs = _reduce_chunk(buf.at[j], accs)

        stage[...] = (accs[0] + accs[1]) + (accs[2] + accs[3])
        pltpu.sync_copy(stage, out_hbm.at[wid])

    return k(gt)


def _tc_sum_kernel(x_ref, out_ref, acc_ref):
    i = pl.program_id(0)

    @pl.when(i == 0)
    def _():
        acc_ref[...] = jnp.zeros_like(acc_ref)

    acc_ref[...] += jnp.sum(
        x_ref[...], axis=(0, 1), dtype=jnp.int32).reshape(1, _D)

    @pl.when(i == pl.num_programs(0) - 1)
    def _():
        out_ref[...] = jnp.sum(
            acc_ref[...], dtype=jnp.int32).reshape(1, 1)


def _tc_partial_sum(gt):
    return pl.pallas_call(
        _tc_sum_kernel,
        grid=(_SC_B0,),
        in_specs=[pl.BlockSpec((1, _H, _D), lambda i: (i, 0, 0))],
        out_specs=pl.BlockSpec((1, 1), lambda i: (0, 0)),
        out_shape=jax.ShapeDtypeStruct((1, 1), jnp.int32),
        scratch_shapes=[pltpu.VMEM((1, _D), jnp.int32)],
    )(gt)


def kernel(gt):
    sc_partials = _sc_partial_sums(gt)
    tc_sum = _tc_partial_sum(gt)
    count1 = jnp.sum(sc_partials) + tc_sum[0, 0]
    sizes = jnp.stack(
        [jnp.int32(_N) - count1, count1]).astype(jnp.float32)
    lowbound = (sizes * (1.0 - _EPS)).astype(jnp.int32)
    highbound = (sizes * (1.0 + _EPS)).astype(jnp.int32)
    return jnp.stack((lowbound, highbound))


# speedup vs baseline: 1.1284x; 1.0551x over previous
import functools

import jax
import jax.numpy as jnp
from jax import lax
from jax.experimental import pallas as pl
from jax.experimental.pallas import tpu as pltpu
from jax.experimental.pallas import tpu_sc as plsc

_EPS = 0.1
_B, _H, _D = 8, 512, 512
_N = _B * _H * _D

_SC_B0 = 6
_SC_N = (_B - _SC_B0) * _H * _D
_NC, _NS = 2, 16
_NW = _NC * _NS
_PER_W = _SC_N // _NW
_LANES = 16
_ROWS_PER_TILE = _PER_W // _D
_TILES_PER_B = _H // _ROWS_PER_TILE
_NCHUNK = 4
_ROWS_PER_CHUNK = _ROWS_PER_TILE // _NCHUNK


def _reduce_chunk(buf, accs):
    def body(i, carry):
        a0, a1, a2, a3 = carry
        for u in range(0, _D // _LANES, 4):
            a0 = a0 + buf[i, pl.ds((u + 0) * _LANES, _LANES)]
            a1 = a1 + buf[i, pl.ds((u + 1) * _LANES, _LANES)]
            a2 = a2 + buf[i, pl.ds((u + 2) * _LANES, _LANES)]
            a3 = a3 + buf[i, pl.ds((u + 3) * _LANES, _LANES)]
        return (a0, a1, a2, a3)

    return lax.fori_loop(0, _ROWS_PER_CHUNK, body, accs)


def _sc_partial_sums(gt):
    mesh = plsc.VectorSubcoreMesh(core_axis_name="c", subcore_axis_name="s")

    @functools.partial(
        pl.kernel,
        out_type=jax.ShapeDtypeStruct((_NW, _LANES), jnp.int32),
        mesh=mesh,
        scratch_types=[
            pltpu.VMEM((_NCHUNK, _ROWS_PER_CHUNK, _D), jnp.int32),
            pltpu.VMEM((_LANES,), jnp.int32),
            pltpu.SemaphoreType.DMA,
            pltpu.SemaphoreType.DMA,
            pltpu.SemaphoreType.DMA,
            pltpu.SemaphoreType.DMA,
        ],
    )
    def k(x_hbm, out_hbm, buf, stage, sem0, sem1, sem2, sem3):
        c = lax.axis_index("c")
        s = lax.axis_index("s")
        wid = s * _NC + c
        b = _SC_B0 + wid // _TILES_PER_B
        r0 = (wid % _TILES_PER_B) * _ROWS_PER_TILE

        sems = (sem0, sem1, sem2, sem3)
        zero = jnp.zeros((_LANES,), jnp.int32)
        accs = (zero, zero, zero, zero)

        copies = [
            pltpu.async_copy(
                x_hbm.at[b, pl.ds(r0 + j * _ROWS_PER_CHUNK,
                                  _ROWS_PER_CHUNK)],
                buf.at[j], sems[j])
            for j in range(_NCHUNK)
        ]
        for j in range(_NCHUNK):
            copies[j].wait()
            accs = _reduce_chunk(buf.at[j], accs)

        stage[...] = (accs[0] + accs[1]) + (accs[2] + accs[3])
        pltpu.sync_copy(stage, out_hbm.at[wid])

    return k(gt)


def _tc_sum_kernel(x_ref, out_ref, acc_ref):
    i = pl.program_id(0)

    @pl.when(i == 0)
    def _():
        acc_ref[...] = jnp.zeros_like(acc_ref)

    acc_ref[...] += jnp.sum(
        x_ref[...], axis=(0, 1), dtype=jnp.int32).reshape(1, _D)

    @pl.when(i == pl.num_programs(0) - 1)
    def _():
        out_ref[...] = jnp.sum(
            acc_ref[...], dtype=jnp.int32).reshape(1, 1)


def _tc_partial_sum(gt):
    return pl.pallas_call(
        _tc_sum_kernel,
        grid=(_SC_B0,),
        in_specs=[pl.BlockSpec((1, _H, _D), lambda i: (i, 0, 0))],
        out_specs=pl.BlockSpec((1, 1), lambda i: (0, 0)),
        out_shape=jax.ShapeDtypeStruct((1, 1), jnp.int32),
        scratch_shapes=[pltpu.VMEM((1, _D), jnp.int32)],
    )(gt)


def kernel(gt):
    sc_partials = _sc_partial_sums(gt)
    tc_sum = _tc_partial_sum(gt)
    count1 = jnp.sum(sc_partials) + tc_sum[0, 0]
    sizes = jnp.stack(
        [jnp.int32(_N) - count1, count1]).astype(jnp.float32)
    lowbound = (sizes * (1.0 - _EPS)).astype(jnp.int32)
    highbound = (sizes * (1.0 + _EPS)).astype(jnp.int32)
    return jnp.stack((lowbound, highbound))


# speedup vs baseline: 1.1941x; 1.0583x over previous
import functools

import jax
import jax.numpy as jnp
from jax import lax
from jax.experimental import pallas as pl
from jax.experimental.pallas import tpu as pltpu
from jax.experimental.pallas import tpu_sc as plsc

_EPS = 0.1
_B, _H, _D = 8, 512, 512
_N = _B * _H * _D

_SC_B0 = 6
_SC_N = (_B - _SC_B0) * _H * _D
_NC, _NS = 2, 16
_NW = _NC * _NS
_PER_W = _SC_N // _NW
_LANES = 16
_ROWS_PER_TILE = _PER_W // _D
_TILES_PER_B = _H // _ROWS_PER_TILE
_NCHUNK = 4
_ROWS_PER_CHUNK = _ROWS_PER_TILE // _NCHUNK


def _reduce_chunk(buf, accs):
    def body(i, carry):
        a0, a1, a2, a3 = carry
        for u in range(0, _D // _LANES, 4):
            a0 = a0 + buf[i, pl.ds((u + 0) * _LANES, _LANES)]
            a1 = a1 + buf[i, pl.ds((u + 1) * _LANES, _LANES)]
            a2 = a2 + buf[i, pl.ds((u + 2) * _LANES, _LANES)]
            a3 = a3 + buf[i, pl.ds((u + 3) * _LANES, _LANES)]
        return (a0, a1, a2, a3)

    return lax.fori_loop(0, _ROWS_PER_CHUNK, body, accs)


def _sc_partial_sums(gt):
    mesh = plsc.VectorSubcoreMesh(core_axis_name="c", subcore_axis_name="s")

    @functools.partial(
        pl.kernel,
        out_type=jax.ShapeDtypeStruct((_NW, _LANES), jnp.int32),
        mesh=mesh,
        scratch_types=[
            pltpu.VMEM((_NCHUNK, _ROWS_PER_CHUNK, _D), jnp.int32),
            pltpu.VMEM((_LANES,), jnp.int32),
            pltpu.SemaphoreType.DMA,
            pltpu.SemaphoreType.DMA,
            pltpu.SemaphoreType.DMA,
            pltpu.SemaphoreType.DMA,
        ],
    )
    def k(x_hbm, out_hbm, buf, stage, sem0, sem1, sem2, sem3):
        c = lax.axis_index("c")
        s = lax.axis_index("s")
        wid = s * _NC + c
        b = _SC_B0 + wid // _TILES_PER_B
        r0 = (wid % _TILES_PER_B) * _ROWS_PER_TILE

        sems = (sem0, sem1, sem2, sem3)
        zero = jnp.zeros((_LANES,), jnp.int32)
        accs = (zero, zero, zero, zero)

        copies = [
            pltpu.async_copy(
                x_hbm.at[b, pl.ds(r0 + j * _ROWS_PER_CHUNK,
                                  _ROWS_PER_CHUNK)],
                buf.at[j], sems[j])
            for j in range(_NCHUNK)
        ]
        for j in range(_NCHUNK):
            copies[j].wait()
            accs = _reduce_chunk(buf.at[j], accs)

        stage[...] = (accs[0] + accs[1]) + (accs[2] + accs[3])
        pltpu.sync_copy(stage, out_hbm.at[wid])

    return k(gt)


def _tc_sum_kernel(x_ref, out_ref, acc_ref):
    i = pl.program_id(0)

    @pl.when(i == 0)
    def _():
        acc_ref[...] = jnp.zeros_like(acc_ref)

    acc_ref[...] += jnp.sum(
        x_ref[...], axis=(0, 1), dtype=jnp.int32).reshape(1, _D)

    @pl.when(i == pl.num_programs(0) - 1)
    def _():
        out_ref[...] = jnp.sum(
            acc_ref[...], dtype=jnp.int32).reshape(1, 1)


def _tc_partial_sum(gt):
    return pl.pallas_call(
        _tc_sum_kernel,
        grid=(_SC_B0,),
        in_specs=[pl.BlockSpec((1, _H, _D), lambda i: (i, 0, 0))],
        out_specs=pl.BlockSpec((1, 1), lambda i: (0, 0)),
        out_shape=jax.ShapeDtypeStruct((1, 1), jnp.int32),
        scratch_shapes=[pltpu.VMEM((1, _D), jnp.int32)],
    )(gt)


def _combine_kernel(parts_ref, tcsum_ref, out_ref):
    count1 = jnp.sum(parts_ref[...], dtype=jnp.int32) + tcsum_ref[0, 0]
    c1 = count1.astype(jnp.float32)
    c0 = (jnp.int32(_N) - count1).astype(jnp.float32)
    col = lax.broadcasted_iota(jnp.int32, (2, 2), 1)
    row = lax.broadcasted_iota(jnp.int32, (2, 2), 0)
    cnt = jnp.where(col == 0, c0, c1)
    fac = jnp.where(row == 0, jnp.float32(1.0 - _EPS),
                    jnp.float32(1.0 + _EPS))
    out_ref[...] = (cnt * fac).astype(jnp.int32)


def _combine(sc_partials, tc_sum):
    return pl.pallas_call(
        _combine_kernel,
        out_shape=jax.ShapeDtypeStruct((2, 2), jnp.int32),
    )(sc_partials, tc_sum)


def kernel(gt):
    sc_partials = _sc_partial_sums(gt)
    tc_sum = _tc_partial_sum(gt)
    return _combine(sc_partials, tc_sum)
